# baseline (speedup 1.0000x reference)
import functools

import jax
import jax.numpy as jnp
from jax import lax
from jax.experimental import pallas as pl
from jax.experimental.pallas import tpu as pltpu
from jax.experimental.pallas import tpu_sc as plsc

N_ROUTED = 64
NUM_LOGITS = 80
RSF = 2.5
T = 2048
D_MODEL = 1024
D_FF = 256
BLK = 64
NBLK = 128
PAD = BLK * NBLK
NW = 32
ROWS_W = T // NW
CH = 128
NCH = T // CH


def _route_body(x_ref, wr_ref, bias_ref,
                p1_ref, p2_ref, w1_ref, w2_ref, zw_ref, be_ref):
    x = x_ref[...]
    logits = lax.dot_general(
        x, wr_ref[...], (((1,), (1,)), ((), ())),
        preferred_element_type=jnp.float32)
    s = jax.nn.sigmoid(logits)
    sc = s + bias_ref[...]
    col = lax.broadcasted_iota(jnp.int32, (T, NUM_LOGITS), 1)
    m1 = jnp.max(sc, axis=1, keepdims=True)
    i1 = jnp.min(jnp.where(sc == m1, col, NUM_LOGITS), axis=1, keepdims=True)
    sc2 = jnp.where(col == i1, -jnp.inf, sc)
    m2 = jnp.max(sc2, axis=1, keepdims=True)
    i2 = jnp.min(jnp.where(sc2 == m2, col, NUM_LOGITS), axis=1, keepdims=True)
    w1v = jnp.sum(jnp.where(col == i1, s, 0.0), axis=1, keepdims=True)
    w2v = jnp.sum(jnp.where(col == i2, s, 0.0), axis=1, keepdims=True)
    norm = w1v + w2v + 1e-20
    w1v = w1v / norm
    w2v = w2v / norm
    z1 = i1 >= N_ROUTED
    z2 = i2 >= N_ROUTED
    zw_ref[...] = jnp.where(z1, w1v, 0.0) + jnp.where(z2, w2v, 0.0)
    e1 = jnp.where(z1, 0, i1)
    e2 = jnp.where(z2, 0, i2)
    w1_ref[...] = jnp.where(z1, 0.0, w1v) * RSF
    w2_ref[...] = jnp.where(z2, 0.0, w2v) * RSF

    ecol = lax.broadcasted_iota(jnp.int32, (T, N_ROUTED), 1)
    oh1 = (ecol == e1).astype(jnp.float32)
    oh2 = (ecol == e2).astype(jnp.float32)
    oh = oh1 + oh2
    ltri = (lax.broadcasted_iota(jnp.int32, (CH, CH), 0)
            >= lax.broadcasted_iota(jnp.int32, (CH, CH), 1)).astype(jnp.float32)
    parts = [
        lax.dot_general(ltri, oh[c * CH:(c + 1) * CH, :],
                        (((1,), (0,)), ((), ())),
                        preferred_element_type=jnp.float32)
        for c in range(NCH)
    ]
    part1 = jnp.concatenate(parts, axis=0)
    sel = (lax.broadcasted_iota(jnp.int32, (NCH, T), 0)
           == lax.broadcasted_iota(jnp.int32, (NCH, T), 1) // CH
           ).astype(jnp.float32)
    chunk_tot = lax.dot_general(sel, oh, (((1,), (0,)), ((), ())),
                                preferred_element_type=jnp.float32)
    l16s = (lax.broadcasted_iota(jnp.int32, (NCH, NCH), 0)
            > lax.broadcasted_iota(jnp.int32, (NCH, NCH), 1)
            ).astype(jnp.float32)
    carry = lax.dot_general(l16s, chunk_tot, (((1,), (0,)), ((), ())),
                            preferred_element_type=jnp.float32)
    expand = (lax.broadcasted_iota(jnp.int32, (T, NCH), 0) // CH
              == lax.broadcasted_iota(jnp.int32, (T, NCH), 1)
              ).astype(jnp.float32)
    c_incl = part1 + lax.dot_general(expand, carry, (((1,), (0,)), ((), ())),
                                     preferred_element_type=jnp.float32)
    c_excl = c_incl - oh
    rank1 = jnp.sum(c_excl * oh1, axis=1, keepdims=True)
    rank2 = (jnp.sum(c_excl * oh2, axis=1, keepdims=True)
             + (e1 == e2).astype(jnp.float32))

    counts = jnp.sum(oh, axis=0, keepdims=True)
    nblk = jnp.floor((counts + (BLK - 1)) / BLK)
    mlt = (lax.broadcasted_iota(jnp.int32, (N_ROUTED, N_ROUTED), 0)
           < lax.broadcasted_iota(jnp.int32, (N_ROUTED, N_ROUTED), 1)
           ).astype(jnp.float32)
    firstblk = lax.dot_general(nblk, mlt, (((1,), (0,)), ((), ())),
                               preferred_element_type=jnp.float32)
    bstart = firstblk * BLK
    p1_ref[...] = (jnp.sum(bstart * oh1, axis=1, keepdims=True)
                   + rank1).astype(jnp.int32)
    p2_ref[...] = (jnp.sum(bstart * oh2, axis=1, keepdims=True)
                   + rank2).astype(jnp.int32)

    bcol = lax.broadcasted_iota(jnp.int32, (NBLK, 1), 0)
    fb_i = firstblk.astype(jnp.int32)
    nb_i = nblk.astype(jnp.int32)
    ind = (fb_i <= bcol) & (nb_i > 0)
    eids = lax.broadcasted_iota(jnp.int32, (NBLK, N_ROUTED), 1)
    be_ref[...] = jnp.maximum(
        jnp.max(jnp.where(ind, eids, -1), axis=1, keepdims=True), 0)


@functools.lru_cache(maxsize=None)
def _make_sc_kernels():
    mesh = plsc.VectorSubcoreMesh(core_axis_name="c", subcore_axis_name="s")
    scratch = [
        pltpu.VMEM((ROWS_W,), jnp.int32),
        pltpu.VMEM((ROWS_W, D_MODEL), jnp.float32),
        pltpu.SemaphoreType.DMA,
    ]

    @functools.partial(
        pl.kernel,
        out_type=jax.ShapeDtypeStruct((PAD, D_MODEL), jnp.float32),
        mesh=mesh,
        scratch_types=scratch,
    )
    def sc_scatter(x_hbm, p1_hbm, p2_hbm, xs_hbm, idx_v, rows_v, sem):
        wid = lax.axis_index("s") * 2 + lax.axis_index("c")
        base = wid * ROWS_W
        pltpu.sync_copy(x_hbm.at[pl.ds(base, ROWS_W)], rows_v)
        pltpu.sync_copy(p1_hbm.at[wid], idx_v)
        pltpu.async_copy(rows_v, xs_hbm.at[idx_v], sem).wait()
        pltpu.sync_copy(p2_hbm.at[wid], idx_v)
        pltpu.async_copy(rows_v, xs_hbm.at[idx_v], sem).wait()

    @functools.partial(
        pl.kernel,
        out_type=(jax.ShapeDtypeStruct((T, D_MODEL), jnp.float32),
                  jax.ShapeDtypeStruct((T, D_MODEL), jnp.float32)),
        mesh=mesh,
        scratch_types=scratch,
    )
    def sc_gather(os_hbm, p1_hbm, p2_hbm, po1_hbm, po2_hbm, idx_v, rows_v, sem):
        wid = lax.axis_index("s") * 2 + lax.axis_index("c")
        base = wid * ROWS_W
        pltpu.sync_copy(p1_hbm.at[wid], idx_v)
        pltpu.async_copy(os_hbm.at[idx_v], rows_v, sem).wait()
        pltpu.sync_copy(rows_v, po1_hbm.at[pl.ds(base, ROWS_W)])
        pltpu.sync_copy(p2_hbm.at[wid], idx_v)
        pltpu.async_copy(os_hbm.at[idx_v], rows_v, sem).wait()
        pltpu.sync_copy(rows_v, po2_hbm.at[pl.ds(base, ROWS_W)])

    return sc_scatter, sc_gather


def _gmm_body(be_ref, xs_ref, w1_ref, w3_ref, w2_ref, out_ref):
    xb = xs_ref[...].astype(jnp.bfloat16)
    a = lax.dot_general(xb, w1_ref[0], (((1,), (1,)), ((), ())),
                        preferred_element_type=jnp.float32)
    c = lax.dot_general(xb, w3_ref[0], (((1,), (1,)), ((), ())),
                        preferred_element_type=jnp.float32)
    h = ((a * jax.nn.sigmoid(a)) * c).astype(jnp.bfloat16)
    out_ref[...] = lax.dot_general(h, w2_ref[0], (((1,), (1,)), ((), ())),
                                   preferred_element_type=jnp.float32)


def _combine_body(po1_ref, po2_ref, x_ref, w1_ref, w2_ref, zw_ref, y_ref):
    y_ref[...] = (w1_ref[...] * po1_ref[...] + w2_ref[...] * po2_ref[...]
                  + zw_ref[...] * x_ref[...])


def kernel(hidden_states, W_router, correction_bias, w1, w3, w2,
           num_global_tokens, max_num_tokens_per_gpu):
    x = hidden_states

    p1, p2, wt1, wt2, zw, be = pl.pallas_call(
        _route_body,
        out_shape=(
            jax.ShapeDtypeStruct((T, 1), jnp.int32),
            jax.ShapeDtypeStruct((T, 1), jnp.int32),
            jax.ShapeDtypeStruct((T, 1), jnp.float32),
            jax.ShapeDtypeStruct((T, 1), jnp.float32),
            jax.ShapeDtypeStruct((T, 1), jnp.float32),
            jax.ShapeDtypeStruct((NBLK, 1), jnp.int32),
        ),
    )(x, W_router, correction_bias.reshape(1, NUM_LOGITS))

    p1w = p1.reshape(NW, ROWS_W)
    p2w = p2.reshape(NW, ROWS_W)
    be1 = be.reshape(NBLK)

    sc_scatter, sc_gather = _make_sc_kernels()
    xs = sc_scatter(x, p1w, p2w)

    os = pl.pallas_call(
        _gmm_body,
        grid_spec=pltpu.PrefetchScalarGridSpec(
            num_scalar_prefetch=1,
            grid=(NBLK,),
            in_specs=[
                pl.BlockSpec((BLK, D_MODEL), lambda b, be_s: (b, 0)),
                pl.BlockSpec((1, D_FF, D_MODEL), lambda b, be_s: (be_s[b], 0, 0)),
                pl.BlockSpec((1, D_FF, D_MODEL), lambda b, be_s: (be_s[b], 0, 0)),
                pl.BlockSpec((1, D_MODEL, D_FF), lambda b, be_s: (be_s[b], 0, 0)),
            ],
            out_specs=pl.BlockSpec((BLK, D_MODEL), lambda b, be_s: (b, 0)),
        ),
        out_shape=jax.ShapeDtypeStruct((PAD, D_MODEL), jnp.float32),
    )(be1, xs, w1.astype(jnp.bfloat16), w3.astype(jnp.bfloat16),
      w2.astype(jnp.bfloat16))

    po1, po2 = sc_gather(os, p1w, p2w)

    y = pl.pallas_call(
        _combine_body,
        grid=(T // 128,),
        in_specs=[
            pl.BlockSpec((128, D_MODEL), lambda i: (i, 0)),
            pl.BlockSpec((128, D_MODEL), lambda i: (i, 0)),
            pl.BlockSpec((128, D_MODEL), lambda i: (i, 0)),
            pl.BlockSpec((128, 1), lambda i: (i, 0)),
            pl.BlockSpec((128, 1), lambda i: (i, 0)),
            pl.BlockSpec((128, 1), lambda i: (i, 0)),
        ],
        out_specs=pl.BlockSpec((128, D_MODEL), lambda i: (i, 0)),
        out_shape=jax.ShapeDtypeStruct((T, D_MODEL), jnp.float32),
    )(po1, po2, x, wt1, wt2, zw)
    return y


# speedup vs baseline: 1.4130x; 1.4130x over previous
import functools

import jax
import jax.numpy as jnp
from jax import lax
from jax.experimental import pallas as pl
from jax.experimental.pallas import tpu as pltpu
from jax.experimental.pallas import tpu_sc as plsc

N_ROUTED = 64
NUM_LOGITS = 80
RSF = 2.5
T = 2048
D_MODEL = 1024
D_FF = 256
BLK = 64
NBLK = 128
PAD = BLK * NBLK
NW = 32
ROWS_W = T // NW
CH = 128
NCH = T // CH


def _route_body(x_ref, wr_ref, bias_ref,
                p1_ref, p2_ref, w1_ref, w2_ref, zw_ref, be_ref):
    x = x_ref[...]
    logits = lax.dot_general(
        x, wr_ref[...], (((1,), (1,)), ((), ())),
        preferred_element_type=jnp.float32)
    s = jax.nn.sigmoid(logits)
    sc = s + bias_ref[...]
    col = lax.broadcasted_iota(jnp.int32, (T, NUM_LOGITS), 1)
    m1 = jnp.max(sc, axis=1, keepdims=True)
    i1 = jnp.min(jnp.where(sc == m1, col, NUM_LOGITS), axis=1, keepdims=True)
    sc2 = jnp.where(col == i1, -jnp.inf, sc)
    m2 = jnp.max(sc2, axis=1, keepdims=True)
    i2 = jnp.min(jnp.where(sc2 == m2, col, NUM_LOGITS), axis=1, keepdims=True)
    w1v = jnp.sum(jnp.where(col == i1, s, 0.0), axis=1, keepdims=True)
    w2v = jnp.sum(jnp.where(col == i2, s, 0.0), axis=1, keepdims=True)
    norm = w1v + w2v + 1e-20
    w1v = w1v / norm
    w2v = w2v / norm
    z1 = i1 >= N_ROUTED
    z2 = i2 >= N_ROUTED
    zw_ref[...] = jnp.where(z1, w1v, 0.0) + jnp.where(z2, w2v, 0.0)
    e1 = jnp.where(z1, 0, i1)
    e2 = jnp.where(z2, 0, i2)
    w1_ref[...] = jnp.where(z1, 0.0, w1v) * RSF
    w2_ref[...] = jnp.where(z2, 0.0, w2v) * RSF

    ecol = lax.broadcasted_iota(jnp.int32, (T, N_ROUTED), 1)
    oh1 = (ecol == e1).astype(jnp.float32)
    oh2 = (ecol == e2).astype(jnp.float32)
    oh = oh1 + oh2
    ltri = (lax.broadcasted_iota(jnp.int32, (CH, CH), 0)
            >= lax.broadcasted_iota(jnp.int32, (CH, CH), 1)).astype(jnp.float32)
    parts = [
        lax.dot_general(ltri, oh[c * CH:(c + 1) * CH, :],
                        (((1,), (0,)), ((), ())),
                        preferred_element_type=jnp.float32)
        for c in range(NCH)
    ]
    part1 = jnp.concatenate(parts, axis=0)
    sel = (lax.broadcasted_iota(jnp.int32, (NCH, T), 0)
           == lax.broadcasted_iota(jnp.int32, (NCH, T), 1) // CH
           ).astype(jnp.float32)
    chunk_tot = lax.dot_general(sel, oh, (((1,), (0,)), ((), ())),
                                preferred_element_type=jnp.float32)
    l16s = (lax.broadcasted_iota(jnp.int32, (NCH, NCH), 0)
            > lax.broadcasted_iota(jnp.int32, (NCH, NCH), 1)
            ).astype(jnp.float32)
    carry = lax.dot_general(l16s, chunk_tot, (((1,), (0,)), ((), ())),
                            preferred_element_type=jnp.float32)
    expand = (lax.broadcasted_iota(jnp.int32, (T, NCH), 0) // CH
              == lax.broadcasted_iota(jnp.int32, (T, NCH), 1)
              ).astype(jnp.float32)
    c_incl = part1 + lax.dot_general(expand, carry, (((1,), (0,)), ((), ())),
                                     preferred_element_type=jnp.float32)
    c_excl = c_incl - oh
    rank1 = jnp.sum(c_excl * oh1, axis=1, keepdims=True)
    rank2 = (jnp.sum(c_excl * oh2, axis=1, keepdims=True)
             + (e1 == e2).astype(jnp.float32))

    counts = jnp.sum(oh, axis=0, keepdims=True)
    nblk = jnp.floor((counts + (BLK - 1)) / BLK)
    mlt = (lax.broadcasted_iota(jnp.int32, (N_ROUTED, N_ROUTED), 0)
           < lax.broadcasted_iota(jnp.int32, (N_ROUTED, N_ROUTED), 1)
           ).astype(jnp.float32)
    firstblk = lax.dot_general(nblk, mlt, (((1,), (0,)), ((), ())),
                               preferred_element_type=jnp.float32)
    bstart = firstblk * BLK
    p1_ref[...] = (jnp.sum(bstart * oh1, axis=1, keepdims=True)
                   + rank1).astype(jnp.int32)
    p2_ref[...] = (jnp.sum(bstart * oh2, axis=1, keepdims=True)
                   + rank2).astype(jnp.int32)

    bcol = lax.broadcasted_iota(jnp.int32, (NBLK, 1), 0)
    fb_i = firstblk.astype(jnp.int32)
    nb_i = nblk.astype(jnp.int32)
    ind = (fb_i <= bcol) & (nb_i > 0)
    eids = lax.broadcasted_iota(jnp.int32, (NBLK, N_ROUTED), 1)
    bexp = jnp.maximum(
        jnp.max(jnp.where(ind, eids, -1), axis=1, keepdims=True), 0)
    total_blocks = jnp.sum(nblk, axis=1, keepdims=True).astype(jnp.int32)
    used = (bcol < total_blocks).astype(jnp.int32)
    be_ref[...] = jnp.concatenate([bexp, used], axis=1)


@functools.lru_cache(maxsize=None)
def _make_sc_kernels():
    mesh = plsc.VectorSubcoreMesh(core_axis_name="c", subcore_axis_name="s")
    scratch = [
        pltpu.VMEM((ROWS_W,), jnp.int32),
        pltpu.VMEM((ROWS_W, D_MODEL), jnp.float32),
        pltpu.SemaphoreType.DMA,
    ]

    @functools.partial(
        pl.kernel,
        out_type=jax.ShapeDtypeStruct((PAD, D_MODEL), jnp.float32),
        mesh=mesh,
        scratch_types=scratch,
    )
    def sc_scatter(x_hbm, p1_hbm, p2_hbm, xs_hbm, idx_v, rows_v, sem):
        wid = lax.axis_index("s") * 2 + lax.axis_index("c")
        base = wid * ROWS_W
        pltpu.sync_copy(x_hbm.at[pl.ds(base, ROWS_W)], rows_v)
        pltpu.sync_copy(p1_hbm.at[wid], idx_v)
        pltpu.async_copy(rows_v, xs_hbm.at[idx_v], sem).wait()
        pltpu.sync_copy(p2_hbm.at[wid], idx_v)
        pltpu.async_copy(rows_v, xs_hbm.at[idx_v], sem).wait()

    @functools.partial(
        pl.kernel,
        out_type=(jax.ShapeDtypeStruct((T, D_MODEL), jnp.float32),
                  jax.ShapeDtypeStruct((T, D_MODEL), jnp.float32)),
        mesh=mesh,
        scratch_types=scratch,
    )
    def sc_gather(os_hbm, p1_hbm, p2_hbm, po1_hbm, po2_hbm, idx_v, rows_v, sem):
        wid = lax.axis_index("s") * 2 + lax.axis_index("c")
        base = wid * ROWS_W
        pltpu.sync_copy(p1_hbm.at[wid], idx_v)
        pltpu.async_copy(os_hbm.at[idx_v], rows_v, sem).wait()
        pltpu.sync_copy(rows_v, po1_hbm.at[pl.ds(base, ROWS_W)])
        pltpu.sync_copy(p2_hbm.at[wid], idx_v)
        pltpu.async_copy(os_hbm.at[idx_v], rows_v, sem).wait()
        pltpu.sync_copy(rows_v, po2_hbm.at[pl.ds(base, ROWS_W)])

    return sc_scatter, sc_gather


def _gmm_body(be_ref, xs_ref, w1_ref, w3_ref, w2_ref, out_ref):
    b = pl.program_id(0)

    @pl.when(be_ref[1, b] == 1)
    def _compute():
        xb = xs_ref[...]
        a = lax.dot_general(xb, w1_ref[0], (((1,), (1,)), ((), ())),
                            preferred_element_type=jnp.float32)
        c = lax.dot_general(xb, w3_ref[0], (((1,), (1,)), ((), ())),
                            preferred_element_type=jnp.float32)
        h = (a * jax.nn.sigmoid(a)) * c
        out_ref[...] = lax.dot_general(
            h, w2_ref[0], (((1,), (1,)), ((), ())),
            preferred_element_type=jnp.float32)


def _combine_body(po1_ref, po2_ref, x_ref, w1_ref, w2_ref, zw_ref, y_ref):
    y_ref[...] = (w1_ref[...] * po1_ref[...] + w2_ref[...] * po2_ref[...]
                  + zw_ref[...] * x_ref[...])


def kernel(hidden_states, W_router, correction_bias, w1, w3, w2,
           num_global_tokens, max_num_tokens_per_gpu):
    x = hidden_states

    p1, p2, wt1, wt2, zw, be = pl.pallas_call(
        _route_body,
        out_shape=(
            jax.ShapeDtypeStruct((T, 1), jnp.int32),
            jax.ShapeDtypeStruct((T, 1), jnp.int32),
            jax.ShapeDtypeStruct((T, 1), jnp.float32),
            jax.ShapeDtypeStruct((T, 1), jnp.float32),
            jax.ShapeDtypeStruct((T, 1), jnp.float32),
            jax.ShapeDtypeStruct((NBLK, 2), jnp.int32),
        ),
    )(x, W_router, correction_bias.reshape(1, NUM_LOGITS))

    p1w = p1.reshape(NW, ROWS_W)
    p2w = p2.reshape(NW, ROWS_W)
    be2 = be.T

    sc_scatter, sc_gather = _make_sc_kernels()
    xs = sc_scatter(x, p1w, p2w)

    os = pl.pallas_call(
        _gmm_body,
        grid_spec=pltpu.PrefetchScalarGridSpec(
            num_scalar_prefetch=1,
            grid=(NBLK,),
            in_specs=[
                pl.BlockSpec((BLK, D_MODEL), lambda b, be_s: (b, 0)),
                pl.BlockSpec((1, D_FF, D_MODEL),
                             lambda b, be_s: (be_s[0, b], 0, 0)),
                pl.BlockSpec((1, D_FF, D_MODEL),
                             lambda b, be_s: (be_s[0, b], 0, 0)),
                pl.BlockSpec((1, D_MODEL, D_FF),
                             lambda b, be_s: (be_s[0, b], 0, 0)),
            ],
            out_specs=pl.BlockSpec((BLK, D_MODEL), lambda b, be_s: (b, 0)),
        ),
        out_shape=jax.ShapeDtypeStruct((PAD, D_MODEL), jnp.float32),
    )(be2, xs, w1, w3, w2)

    po1, po2 = sc_gather(os, p1w, p2w)

    y = pl.pallas_call(
        _combine_body,
        grid=(T // 128,),
        in_specs=[
            pl.BlockSpec((128, D_MODEL), lambda i: (i, 0)),
            pl.BlockSpec((128, D_MODEL), lambda i: (i, 0)),
            pl.BlockSpec((128, D_MODEL), lambda i: (i, 0)),
            pl.BlockSpec((128, 1), lambda i: (i, 0)),
            pl.BlockSpec((128, 1), lambda i: (i, 0)),
            pl.BlockSpec((128, 1), lambda i: (i, 0)),
        ],
        out_specs=pl.BlockSpec((128, D_MODEL), lambda i: (i, 0)),
        out_shape=jax.ShapeDtypeStruct((T, D_MODEL), jnp.float32),
    )(po1, po2, x, wt1, wt2, zw)
    return y


# speedup vs baseline: 1.5822x; 1.1197x over previous
import functools

import jax
import jax.numpy as jnp
from jax import lax
from jax.experimental import pallas as pl
from jax.experimental.pallas import tpu as pltpu
from jax.experimental.pallas import tpu_sc as plsc

N_ROUTED = 64
NUM_LOGITS = 80
RSF = 2.5
T = 2048
D_MODEL = 1024
D_FF = 256
BLK = 64
NBLK = 128
PAD = BLK * NBLK
NW = 32
ROWS_W = T // NW
CH = 128
NCH = T // CH


def _route_body(x_ref, wr_ref, bias_ref,
                p1_ref, p2_ref, w1_ref, w2_ref, zw_ref, be_ref):
    x = x_ref[...]
    logits = lax.dot_general(
        x, wr_ref[...], (((1,), (1,)), ((), ())),
        preferred_element_type=jnp.float32)
    s = jax.nn.sigmoid(logits)
    sc = s + bias_ref[...]
    col = lax.broadcasted_iota(jnp.int32, (T, NUM_LOGITS), 1)
    m1 = jnp.max(sc, axis=1, keepdims=True)
    i1 = jnp.min(jnp.where(sc == m1, col, NUM_LOGITS), axis=1, keepdims=True)
    sc2 = jnp.where(col == i1, -jnp.inf, sc)
    m2 = jnp.max(sc2, axis=1, keepdims=True)
    i2 = jnp.min(jnp.where(sc2 == m2, col, NUM_LOGITS), axis=1, keepdims=True)
    w1v = jnp.sum(jnp.where(col == i1, s, 0.0), axis=1, keepdims=True)
    w2v = jnp.sum(jnp.where(col == i2, s, 0.0), axis=1, keepdims=True)
    norm = w1v + w2v + 1e-20
    w1v = w1v / norm
    w2v = w2v / norm
    z1 = i1 >= N_ROUTED
    z2 = i2 >= N_ROUTED
    zw_ref[...] = jnp.where(z1, w1v, 0.0) + jnp.where(z2, w2v, 0.0)
    e1 = jnp.where(z1, 0, i1)
    e2 = jnp.where(z2, 0, i2)
    w1_ref[...] = jnp.where(z1, 0.0, w1v) * RSF
    w2_ref[...] = jnp.where(z2, 0.0, w2v) * RSF

    ecol = lax.broadcasted_iota(jnp.int32, (T, N_ROUTED), 1)
    oh1 = (ecol == e1).astype(jnp.float32)
    oh2 = (ecol == e2).astype(jnp.float32)
    oh = oh1 + oh2
    ltri = (lax.broadcasted_iota(jnp.int32, (CH, CH), 0)
            >= lax.broadcasted_iota(jnp.int32, (CH, CH), 1)).astype(jnp.float32)
    parts = [
        lax.dot_general(ltri, oh[c * CH:(c + 1) * CH, :],
                        (((1,), (0,)), ((), ())),
                        preferred_element_type=jnp.float32)
        for c in range(NCH)
    ]
    part1 = jnp.concatenate(parts, axis=0)
    sel = (lax.broadcasted_iota(jnp.int32, (NCH, T), 0)
           == lax.broadcasted_iota(jnp.int32, (NCH, T), 1) // CH
           ).astype(jnp.float32)
    chunk_tot = lax.dot_general(sel, oh, (((1,), (0,)), ((), ())),
                                preferred_element_type=jnp.float32)
    l16s = (lax.broadcasted_iota(jnp.int32, (NCH, NCH), 0)
            > lax.broadcasted_iota(jnp.int32, (NCH, NCH), 1)
            ).astype(jnp.float32)
    carry = lax.dot_general(l16s, chunk_tot, (((1,), (0,)), ((), ())),
                            preferred_element_type=jnp.float32)
    expand = (lax.broadcasted_iota(jnp.int32, (T, NCH), 0) // CH
              == lax.broadcasted_iota(jnp.int32, (T, NCH), 1)
              ).astype(jnp.float32)
    c_incl = part1 + lax.dot_general(expand, carry, (((1,), (0,)), ((), ())),
                                     preferred_element_type=jnp.float32)
    c_excl = c_incl - oh
    rank1 = jnp.sum(c_excl * oh1, axis=1, keepdims=True)
    rank2 = (jnp.sum(c_excl * oh2, axis=1, keepdims=True)
             + (e1 == e2).astype(jnp.float32))

    counts = jnp.sum(oh, axis=0, keepdims=True)
    nblk = jnp.floor((counts + (BLK - 1)) / BLK)
    mlt = (lax.broadcasted_iota(jnp.int32, (N_ROUTED, N_ROUTED), 0)
           < lax.broadcasted_iota(jnp.int32, (N_ROUTED, N_ROUTED), 1)
           ).astype(jnp.float32)
    firstblk = lax.dot_general(nblk, mlt, (((1,), (0,)), ((), ())),
                               preferred_element_type=jnp.float32)
    bstart = firstblk * BLK
    p1_ref[...] = (jnp.sum(bstart * oh1, axis=1, keepdims=True)
                   + rank1).astype(jnp.int32)
    p2_ref[...] = (jnp.sum(bstart * oh2, axis=1, keepdims=True)
                   + rank2).astype(jnp.int32)

    bcol = lax.broadcasted_iota(jnp.int32, (NBLK, 1), 0)
    fb_i = firstblk.astype(jnp.int32)
    nb_i = nblk.astype(jnp.int32)
    ind = (fb_i <= bcol) & (nb_i > 0)
    eids = lax.broadcasted_iota(jnp.int32, (NBLK, N_ROUTED), 1)
    bexp = jnp.maximum(
        jnp.max(jnp.where(ind, eids, -1), axis=1, keepdims=True), 0)
    total_blocks = jnp.sum(nblk, axis=1, keepdims=True).astype(jnp.int32)
    used = (bcol < total_blocks).astype(jnp.int32)
    redirect = jnp.minimum(bcol, total_blocks - 1)
    be_ref[...] = jnp.concatenate([bexp, used, redirect], axis=1)


@functools.lru_cache(maxsize=None)
def _make_sc_kernels():
    mesh = plsc.VectorSubcoreMesh(core_axis_name="c", subcore_axis_name="s")
    scratch = [
        pltpu.VMEM((ROWS_W,), jnp.int32),
        pltpu.VMEM((ROWS_W, D_MODEL), jnp.float32),
        pltpu.SemaphoreType.DMA,
    ]

    @functools.partial(
        pl.kernel,
        out_type=jax.ShapeDtypeStruct((PAD, D_MODEL), jnp.float32),
        mesh=mesh,
        scratch_types=scratch,
    )
    def sc_scatter(x_hbm, p1_hbm, p2_hbm, xs_hbm, idx_v, rows_v, sem):
        wid = lax.axis_index("s") * 2 + lax.axis_index("c")
        base = wid * ROWS_W
        pltpu.sync_copy(x_hbm.at[pl.ds(base, ROWS_W)], rows_v)
        pltpu.sync_copy(p1_hbm.at[wid], idx_v)
        pltpu.async_copy(rows_v, xs_hbm.at[idx_v], sem).wait()
        pltpu.sync_copy(p2_hbm.at[wid], idx_v)
        pltpu.async_copy(rows_v, xs_hbm.at[idx_v], sem).wait()

    @functools.partial(
        pl.kernel,
        out_type=(jax.ShapeDtypeStruct((T, D_MODEL), jnp.float32),
                  jax.ShapeDtypeStruct((T, D_MODEL), jnp.float32)),
        mesh=mesh,
        scratch_types=scratch,
    )
    def sc_gather(os_hbm, p1_hbm, p2_hbm, po1_hbm, po2_hbm, idx_v, rows_v, sem):
        wid = lax.axis_index("s") * 2 + lax.axis_index("c")
        base = wid * ROWS_W
        pltpu.sync_copy(p1_hbm.at[wid], idx_v)
        pltpu.async_copy(os_hbm.at[idx_v], rows_v, sem).wait()
        pltpu.sync_copy(rows_v, po1_hbm.at[pl.ds(base, ROWS_W)])
        pltpu.sync_copy(p2_hbm.at[wid], idx_v)
        pltpu.async_copy(os_hbm.at[idx_v], rows_v, sem).wait()
        pltpu.sync_copy(rows_v, po2_hbm.at[pl.ds(base, ROWS_W)])

    return sc_scatter, sc_gather


def _gmm_body(be_ref, xs_ref, w1_ref, w3_ref, w2_ref, out_ref):
    b = pl.program_id(0)

    @pl.when(be_ref[1, b] == 1)
    def _compute():
        xb = xs_ref[...]
        a = lax.dot_general(xb, w1_ref[0], (((1,), (1,)), ((), ())),
                            preferred_element_type=jnp.float32)
        c = lax.dot_general(xb, w3_ref[0], (((1,), (1,)), ((), ())),
                            preferred_element_type=jnp.float32)
        h = (a * jax.nn.sigmoid(a)) * c
        out_ref[...] = lax.dot_general(
            h, w2_ref[0], (((1,), (1,)), ((), ())),
            preferred_element_type=jnp.float32)


def _combine_body(po1_ref, po2_ref, x_ref, w1_ref, w2_ref, zw_ref, y_ref):
    y_ref[...] = (w1_ref[...] * po1_ref[...] + w2_ref[...] * po2_ref[...]
                  + zw_ref[...] * x_ref[...])


def kernel(hidden_states, W_router, correction_bias, w1, w3, w2,
           num_global_tokens, max_num_tokens_per_gpu):
    x = hidden_states

    p1, p2, wt1, wt2, zw, be = pl.pallas_call(
        _route_body,
        out_shape=(
            jax.ShapeDtypeStruct((T, 1), jnp.int32),
            jax.ShapeDtypeStruct((T, 1), jnp.int32),
            jax.ShapeDtypeStruct((T, 1), jnp.float32),
            jax.ShapeDtypeStruct((T, 1), jnp.float32),
            jax.ShapeDtypeStruct((T, 1), jnp.float32),
            jax.ShapeDtypeStruct((NBLK, 3), jnp.int32),
        ),
    )(x, W_router, correction_bias.reshape(1, NUM_LOGITS))

    p1w = p1.reshape(NW, ROWS_W)
    p2w = p2.reshape(NW, ROWS_W)
    be2 = be.T

    sc_scatter, sc_gather = _make_sc_kernels()
    xs = sc_scatter(x, p1w, p2w)

    os = pl.pallas_call(
        _gmm_body,
        grid_spec=pltpu.PrefetchScalarGridSpec(
            num_scalar_prefetch=1,
            grid=(NBLK,),
            in_specs=[
                pl.BlockSpec((BLK, D_MODEL), lambda b, be_s: (be_s[2, b], 0)),
                pl.BlockSpec((1, D_FF, D_MODEL),
                             lambda b, be_s: (be_s[0, b], 0, 0)),
                pl.BlockSpec((1, D_FF, D_MODEL),
                             lambda b, be_s: (be_s[0, b], 0, 0)),
                pl.BlockSpec((1, D_MODEL, D_FF),
                             lambda b, be_s: (be_s[0, b], 0, 0)),
            ],
            out_specs=pl.BlockSpec((BLK, D_MODEL),
                                   lambda b, be_s: (be_s[2, b], 0)),
        ),
        out_shape=jax.ShapeDtypeStruct((PAD, D_MODEL), jnp.float32),
    )(be2, xs, w1, w3, w2)

    po1, po2 = sc_gather(os, p1w, p2w)

    y = pl.pallas_call(
        _combine_body,
        grid=(T // 128,),
        in_specs=[
            pl.BlockSpec((128, D_MODEL), lambda i: (i, 0)),
            pl.BlockSpec((128, D_MODEL), lambda i: (i, 0)),
            pl.BlockSpec((128, D_MODEL), lambda i: (i, 0)),
            pl.BlockSpec((128, 1), lambda i: (i, 0)),
            pl.BlockSpec((128, 1), lambda i: (i, 0)),
            pl.BlockSpec((128, 1), lambda i: (i, 0)),
        ],
        out_specs=pl.BlockSpec((128, D_MODEL), lambda i: (i, 0)),
        out_shape=jax.ShapeDtypeStruct((T, D_MODEL), jnp.float32),
    )(po1, po2, x, wt1, wt2, zw)
    return y
